# Initial kernel scaffold; baseline (speedup 1.0000x reference)
import jax
import jax.numpy as jnp
from jax.experimental import pallas as pl


def kernel(point_features, point_coords, Wg1, bg1, Wg2, bg2, Wc1, bc1, Wc2, bc2, Wd1, bd1, Wd2, bd2, Wp1, bp1, Wp2, bp2, ln_g, ln_b):
    raise NotImplementedError("write your pallas kernel here")


# speedup vs baseline: 383.9119x; 383.9119x over previous
import functools

import jax
import jax.numpy as jnp
from jax import lax
from jax.experimental import pallas as pl
from jax.experimental.pallas import tpu as pltpu
from jax.experimental.pallas import tpu_sc as plsc

B = 4
N = 4096
D = 768
NCEN = 256

_HIGH = jax.lax.Precision.DEFAULT


def _fps_body(cx_ref, cy_ref, cz_ref, flat_ref, ccx_ref, ccy_ref, ccz_ref,
              dist_ref):
    cx = cx_ref[...]
    cy = cy_ref[...]
    cz = cz_ref[...]
    ii = lax.broadcasted_iota(jnp.int32, (B, N), 1)
    bb = lax.broadcasted_iota(jnp.int32, (B, 1), 0) * N

    lpx0 = cx[:, 0:1]
    lpy0 = cy[:, 0:1]
    lpz0 = cz[:, 0:1]
    flat_ref[:, 0:1, :] = bb[:, :, None]
    ccx_ref[:, 0:1, :] = lpx0[:, :, None]
    ccy_ref[:, 0:1, :] = lpy0[:, :, None]
    ccz_ref[:, 0:1, :] = lpz0[:, :, None]
    dist_ref[...] = jnp.full((B, N), jnp.inf, dtype=jnp.float32)

    def body(t, carry):
        lpx, lpy, lpz = carry
        dx = cx - lpx
        dy = cy - lpy
        dz = cz - lpz
        d = dx * dx + dy * dy + dz * dz
        dist = jnp.minimum(dist_ref[...], d)
        m = jnp.max(dist, axis=1, keepdims=True)
        cand = jnp.where(dist == m, ii, jnp.int32(2 ** 30))
        nxt = jnp.min(cand, axis=1, keepdims=True)
        msk = ii == nxt
        dist_ref[...] = jnp.where(msk, 0.0, dist)
        nlpx = jnp.sum(jnp.where(msk, cx, 0.0), axis=1, keepdims=True)
        nlpy = jnp.sum(jnp.where(msk, cy, 0.0), axis=1, keepdims=True)
        nlpz = jnp.sum(jnp.where(msk, cz, 0.0), axis=1, keepdims=True)
        flat_ref[:, pl.ds(t, 1), :] = (nxt + bb)[:, :, None]
        ccx_ref[:, pl.ds(t, 1), :] = nlpx[:, :, None]
        ccy_ref[:, pl.ds(t, 1), :] = nlpy[:, :, None]
        ccz_ref[:, pl.ds(t, 1), :] = nlpz[:, :, None]
        return (nlpx, nlpy, nlpz)

    lax.fori_loop(1, NCEN, body, (lpx0, lpy0, lpz0))


def _fps(cx, cy, cz):
    return pl.pallas_call(
        _fps_body,
        out_shape=[
            jax.ShapeDtypeStruct((B, NCEN, 1), jnp.int32),
            jax.ShapeDtypeStruct((B, NCEN, 1), jnp.float32),
            jax.ShapeDtypeStruct((B, NCEN, 1), jnp.float32),
            jax.ShapeDtypeStruct((B, NCEN, 1), jnp.float32),
        ],
        scratch_shapes=[pltpu.VMEM((B, N), jnp.float32)],
    )(cx, cy, cz)


_NW = 32
_ROWS = B * NCEN
_RPW = _ROWS // _NW


def _sc_gather(table, idx):
    mesh = plsc.VectorSubcoreMesh(core_axis_name="c", subcore_axis_name="s")

    @functools.partial(
        pl.kernel,
        mesh=mesh,
        out_type=jax.ShapeDtypeStruct((_ROWS, D), jnp.float32),
        scratch_types=[
            pltpu.VMEM((_RPW,), jnp.int32),
            pltpu.VMEM((_RPW, D), jnp.float32),
            pltpu.SemaphoreType.DMA,
        ],
    )
    def gather_kernel(table_hbm, idx_hbm, out_hbm, idx_v, rows_v, sem):
        wid = lax.axis_index("s") * 2 + lax.axis_index("c")
        base = wid * _RPW
        pltpu.sync_copy(idx_hbm.at[pl.ds(base, _RPW)], idx_v)
        pltpu.async_copy(table_hbm.at[idx_v], rows_v, sem).wait()
        pltpu.sync_copy(rows_v, out_hbm.at[pl.ds(base, _RPW)])

    return gather_kernel(table, idx)


def _dot(a, b):
    return jax.lax.dot_general(a, b, (((1,), (0,)), ((), ())),
                               precision=_HIGH,
                               preferred_element_type=jnp.float32)


def _sigmoid(x):
    return 1.0 / (1.0 + jnp.exp(-x))


def _select_body(ce_ref, ccxc_ref, ccyc_ref, cczc_ref,
                 ccxr_ref, ccyr_ref, cczr_ref,
                 wg1_ref, bg1_ref, wg2_ref, bg2_ref,
                 wc1_ref, bc1_ref, wc2_ref, bc2_ref,
                 wd1_ref, bd1_ref, wd2_ref, bd2_ref,
                 wp1_ref, bp1_ref, wp2_ref, bp2_ref,
                 lng_ref, lnb_ref, out_ref, sel_ref):
    def topk_gather(score, cvec, k, ce_base, sel_base):
        iot = lax.broadcasted_iota(jnp.int32, (cvec, 1), 0)
        for j in range(k):
            m = jnp.max(score)
            idx = jnp.min(jnp.where(score == m, iot, jnp.int32(2 ** 30)))
            sel_ref[pl.ds(sel_base + j, 1), :] = \
                ce_ref[pl.ds(ce_base + idx, 1), :]
            score = jnp.where(iot == idx, -jnp.inf, score)

    for b in range(B):
        ce_b = ce_ref[pl.ds(b * NCEN, NCEN), :]

        h = jnp.maximum(_dot(ce_b, wg1_ref[...]) + bg1_ref[...], 0.0)
        pg = _sigmoid(_dot(h, wg2_ref[...]) + bg2_ref[...])
        z = cczc_ref[b]
        hr = _sigmoid((z - jnp.mean(z)) / 5.0)
        sg = 1.0 + hr * 0.95
        score_g = jnp.mean(pg * sg, axis=1, keepdims=True)
        topk_gather(score_g, NCEN, 16, b * NCEN, 0)

        ce_c = ce_ref[pl.ds(b * NCEN, 128), :]
        h = jnp.maximum(_dot(ce_c, wc1_ref[...]) + bc1_ref[...], 0.0)
        pc = _sigmoid(_dot(h, wc2_ref[...]) + bc2_ref[...])
        score_c = jnp.mean(pc, axis=1, keepdims=True)
        topk_gather(score_c, 128, 16, b * NCEN, 16)

        ce_d = ce_ref[pl.ds(b * NCEN, 64), :]
        h = jnp.maximum(_dot(ce_d, wd1_ref[...]) + bd1_ref[...], 0.0)
        pd = _sigmoid(_dot(h, wd2_ref[...]) + bd2_ref[...])
        xi = ccxc_ref[b, 0:64, :]
        yi = ccyc_ref[b, 0:64, :]
        zi = cczc_ref[b, 0:64, :]
        xj = ccxr_ref[b:b + 1, 0:64]
        yj = ccyr_ref[b:b + 1, 0:64]
        zj = cczr_ref[b:b + 1, 0:64]
        dxx = xi - xj
        dyy = yi - yj
        dzz = zi - zj
        d2 = dxx * dxx + dyy * dyy + dzz * dzz
        dens = jnp.sum(jnp.where(d2 < 0.25, 1.0, 0.0), axis=1, keepdims=True)
        sd = 1.0 + dens / 64.0 * 0.95
        score_d = jnp.mean(pd * sd, axis=1, keepdims=True)
        topk_gather(score_d, 64, 8, b * NCEN, 32)

        sel = sel_ref[...]
        h2 = jnp.maximum(_dot(sel, wp1_ref[...]) + bp1_ref[...], 0.0)
        o = _dot(h2, wp2_ref[...]) + bp2_ref[...]
        mu = jnp.mean(o, axis=1, keepdims=True)
        var = jnp.mean((o - mu) * (o - mu), axis=1, keepdims=True)
        out_ref[b] = (o - mu) / jnp.sqrt(var + 1e-5) * lng_ref[...] \
            + lnb_ref[...]


def _select(ce, ccxc, ccyc, cczc, ccxr, ccyr, cczr, *weights):
    return pl.pallas_call(
        _select_body,
        out_shape=jax.ShapeDtypeStruct((B, 40, D), jnp.float32),
        scratch_shapes=[pltpu.VMEM((40, D), jnp.float32)],
    )(ce, ccxc, ccyc, cczc, ccxr, ccyr, cczr, *weights)


def kernel(point_features, point_coords, Wg1, bg1, Wg2, bg2, Wc1, bc1, Wc2,
           bc2, Wd1, bd1, Wd2, bd2, Wp1, bp1, Wp2, bp2, ln_g, ln_b):
    cx = point_coords[:, :, 0]
    cy = point_coords[:, :, 1]
    cz = point_coords[:, :, 2]

    flat, ccx, ccy, ccz = _fps(cx, cy, cz)

    idx = flat.reshape(_ROWS)
    ce = _sc_gather(point_features.reshape(B * N, D), idx)

    out = _select(
        ce, ccx, ccy, ccz,
        ccx.reshape(B, NCEN), ccy.reshape(B, NCEN), ccz.reshape(B, NCEN),
        Wg1, bg1.reshape(1, -1), Wg2, bg2.reshape(1, -1),
        Wc1, bc1.reshape(1, -1), Wc2, bc2.reshape(1, -1),
        Wd1, bd1.reshape(1, -1), Wd2, bd2.reshape(1, -1),
        Wp1, bp1.reshape(1, -1), Wp2, bp2.reshape(1, -1),
        ln_g.reshape(1, -1), ln_b.reshape(1, -1),
    )
    return out
